# Initial kernel scaffold; baseline (speedup 1.0000x reference)
import jax
import jax.numpy as jnp
from jax.experimental import pallas as pl


def kernel(querys, prompts_key, prompts):
    raise NotImplementedError("write your pallas kernel here")


# speedup vs baseline: 1.7798x; 1.7798x over previous
import jax
import jax.numpy as jnp
from jax import lax
from jax.experimental import pallas as pl
from jax.experimental.pallas import tpu as pltpu
from jax.experimental.pallas import tpu_sc as plsc

POOL = 1000
POOL_PAD = 1024
TOPK = 4
PROMPT_LEN = 8
HIDDEN = 1024
QDIM = 2 * HIDDEN
BATCH = 1024

QBLK = 128
NSTEPS = BATCH // QBLK

NIDX = BATCH * TOPK * PROMPT_LEN
NWORKERS = 32
PER_W = NIDX // NWORKERS
CHUNK = 32
NCHUNK = PER_W // CHUNK


def _topk_body(q_ref, k_ref, sim_ref, idx_ref, loss_ref):
    step = pl.program_id(0)
    q = q_ref[...]
    km = k_ref[...]
    qn = jnp.sqrt(jnp.sum(q * q, axis=1, keepdims=True))
    kn_col = jnp.sqrt(jnp.sum(km * km, axis=1, keepdims=True))
    kn = jnp.sqrt(jnp.sum(km * km, axis=1))[None, :]
    dots = jax.lax.dot_general(
        q, km, dimension_numbers=(((1,), (1,)), ((), ())),
        preferred_element_type=jnp.float32)
    sim = dots / jnp.maximum(qn * kn, 1e-8)

    qnorm = q / jnp.maximum(qn, 1e-12)
    knorm = km / jnp.maximum(kn_col, 1e-12)
    sim2 = jax.lax.dot_general(
        qnorm, knorm, dimension_numbers=(((1,), (1,)), ((), ())),
        preferred_element_type=jnp.float32)
    part = jnp.full((1, 1), -jnp.sum(sim2) / (BATCH * POOL), jnp.float32)

    @pl.when(step == 0)
    def _():
        loss_ref[...] = jnp.zeros((1, 1), jnp.float32)
    loss_ref[...] += part

    col = jax.lax.broadcasted_iota(jnp.int32, (QBLK, POOL_PAD), 1)
    iota8 = jax.lax.broadcasted_iota(jnp.int32, (QBLK, PROMPT_LEN), 1)
    m = jnp.where(col < POOL, sim, -2.0)
    for k in range(TOPK):
        mx = jnp.max(m, axis=1, keepdims=True)
        amax = jnp.min(jnp.where(m == mx, col, 2**30), axis=1, keepdims=True)
        sim_ref[:, pl.ds(k, 1)] = mx
        idx_ref[:, pl.ds(k * PROMPT_LEN, PROMPT_LEN)] = amax * PROMPT_LEN + iota8
        m = jnp.where(col == amax, -3.0, m)


def _topk_call(querys, pk_pad):
    return pl.pallas_call(
        _topk_body,
        grid=(NSTEPS,),
        in_specs=[
            pl.BlockSpec((QBLK, QDIM), lambda i: (i, 0)),
            pl.BlockSpec((POOL_PAD, QDIM), lambda i: (0, 0)),
        ],
        out_specs=[
            pl.BlockSpec((QBLK, TOPK), lambda i: (i, 0)),
            pl.BlockSpec((QBLK, TOPK * PROMPT_LEN), lambda i: (i, 0)),
            pl.BlockSpec((1, 1), lambda i: (0, 0)),
        ],
        out_shape=[
            jax.ShapeDtypeStruct((BATCH, TOPK), jnp.float32),
            jax.ShapeDtypeStruct((BATCH, TOPK * PROMPT_LEN), jnp.int32),
            jax.ShapeDtypeStruct((1, 1), jnp.float32),
        ],
    )(querys, pk_pad)


def _sc_gather(flat, idx):
    mesh = plsc.VectorSubcoreMesh(core_axis_name="c", subcore_axis_name="s")

    @pl.kernel(
        out_type=jax.ShapeDtypeStruct((NIDX, HIDDEN), jnp.float32),
        mesh=mesh,
        scratch_types=[
            pltpu.VMEM((PER_W,), jnp.int32),
            pltpu.VMEM((CHUNK, HIDDEN), jnp.float32),
            pltpu.VMEM((CHUNK, HIDDEN), jnp.float32),
            pltpu.SemaphoreType.DMA,
            pltpu.SemaphoreType.DMA,
        ],
    )
    def gk(flat_hbm, idx_hbm, out_hbm, idx_v, buf0, buf1, sem0, sem1):
        wid = lax.axis_index("s") * 2 + lax.axis_index("c")
        base = wid * PER_W
        pltpu.sync_copy(idx_hbm.at[pl.ds(base, PER_W)], idx_v)
        bufs = (buf0, buf1)
        sems = (sem0, sem1)

        def start_gather(c, p):
            pltpu.make_async_copy(
                flat_hbm.at[idx_v.at[pl.ds(c * CHUNK, CHUNK)]],
                bufs[p], sems[p]).start()

        def wait_gather(p):
            pltpu.make_async_copy(
                flat_hbm.at[idx_v.at[pl.ds(0, CHUNK)]],
                bufs[p], sems[p]).wait()

        start_gather(0, 0)
        start_gather(1, 1)

        @pl.loop(0, NCHUNK, step=2)
        def _(c):
            for p in range(2):
                ch = c + p
                wait_gather(p)
                pltpu.sync_copy(
                    bufs[p], out_hbm.at[pl.ds(base + ch * CHUNK, CHUNK)])
                nxt = ch + 2

                @pl.when(nxt < NCHUNK)
                def _():
                    start_gather(nxt, p)

    return gk(flat, idx)


def kernel(querys, prompts_key, prompts):
    pk_pad = jnp.pad(prompts_key, ((0, POOL_PAD - POOL), (0, 0)))
    sim_topk, idx8, loss = _topk_call(querys, pk_pad)
    flat = prompts.reshape(POOL * PROMPT_LEN, HIDDEN)
    gathered = _sc_gather(flat, idx8.reshape(NIDX))
    selected = gathered.reshape(BATCH, TOPK, PROMPT_LEN, HIDDEN)
    return selected, sim_topk, loss.reshape(())


# speedup vs baseline: 2.0158x; 1.1326x over previous
import jax
import jax.numpy as jnp
from jax import lax
from jax.experimental import pallas as pl
from jax.experimental.pallas import tpu as pltpu
from jax.experimental.pallas import tpu_sc as plsc

POOL = 1000
POOL_PAD = 1024
TOPK = 4
PROMPT_LEN = 8
HIDDEN = 1024
QDIM = 2 * HIDDEN
BATCH = 1024

QBLK = 256
NSTEPS = BATCH // QBLK

NIDX = BATCH * TOPK * PROMPT_LEN
NWORKERS = 32
PER_W = NIDX // NWORKERS
CHUNK = 16
NBUF = 4
NCHUNK = PER_W // CHUNK


def _topk_body(q_ref, k_ref, sim_ref, idx_ref, loss_ref):
    step = pl.program_id(0)
    q = q_ref[...]
    km = k_ref[...]
    qn = jnp.sqrt(jnp.sum(q * q, axis=1, keepdims=True))
    kn_col = jnp.sqrt(jnp.sum(km * km, axis=1, keepdims=True))
    kn = jnp.sqrt(jnp.sum(km * km, axis=1))[None, :]

    dots = jax.lax.dot_general(
        q, km, dimension_numbers=(((1,), (1,)), ((), ())),
        preferred_element_type=jnp.float32)
    sim = dots / jnp.maximum(qn * kn, 1e-8)

    qnorm = q / jnp.maximum(qn, 1e-12)
    knorm = km / jnp.maximum(kn_col, 1e-12)
    sim2 = jax.lax.dot_general(
        qnorm, knorm, dimension_numbers=(((1,), (1,)), ((), ())),
        preferred_element_type=jnp.float32)
    part = jnp.full((1, 1), -jnp.sum(sim2) / (BATCH * POOL), jnp.float32)

    @pl.when(step == 0)
    def _():
        loss_ref[...] = jnp.zeros((1, 1), jnp.float32)
    loss_ref[...] += part

    col = jax.lax.broadcasted_iota(jnp.int32, (QBLK, POOL_PAD), 1)
    iota8 = jax.lax.broadcasted_iota(jnp.int32, (QBLK, PROMPT_LEN), 1)
    m = jnp.where(col < POOL, sim, -2.0)
    for k in range(TOPK):
        mx = jnp.max(m, axis=1, keepdims=True)
        amax = jnp.min(jnp.where(m == mx, col, 2**30), axis=1, keepdims=True)
        sim_ref[:, pl.ds(k, 1)] = mx
        idx_ref[:, pl.ds(k * PROMPT_LEN, PROMPT_LEN)] = amax * PROMPT_LEN + iota8
        m = jnp.where(col == amax, -3.0, m)


def _topk_call(querys, pk_pad):
    return pl.pallas_call(
        _topk_body,
        grid=(NSTEPS,),
        in_specs=[
            pl.BlockSpec((QBLK, QDIM), lambda i: (i, 0)),
            pl.BlockSpec((POOL_PAD, QDIM), lambda i: (0, 0)),
        ],
        out_specs=[
            pl.BlockSpec((QBLK, TOPK), lambda i: (i, 0)),
            pl.BlockSpec((QBLK, TOPK * PROMPT_LEN), lambda i: (i, 0)),
            pl.BlockSpec((1, 1), lambda i: (0, 0)),
        ],
        out_shape=[
            jax.ShapeDtypeStruct((BATCH, TOPK), jnp.float32),
            jax.ShapeDtypeStruct((BATCH, TOPK * PROMPT_LEN), jnp.int32),
            jax.ShapeDtypeStruct((1, 1), jnp.float32),
        ],
    )(querys, pk_pad)


def _sc_gather(flat, idx):
    mesh = plsc.VectorSubcoreMesh(core_axis_name="c", subcore_axis_name="s")

    @pl.kernel(
        out_type=jax.ShapeDtypeStruct((NIDX, HIDDEN), jnp.float32),
        mesh=mesh,
        scratch_types=[
            pltpu.VMEM((PER_W,), jnp.int32),
        ] + [pltpu.VMEM((CHUNK, HIDDEN), jnp.float32)] * NBUF
          + [pltpu.SemaphoreType.DMA] * NBUF
          + [pltpu.SemaphoreType.DMA] * NBUF,
    )
    def gk(flat_hbm, idx_hbm, out_hbm, idx_v, *bufs_sems):
        bufs = bufs_sems[:NBUF]
        gsems = bufs_sems[NBUF:2 * NBUF]
        osems = bufs_sems[2 * NBUF:]
        wid = lax.axis_index("s") * 2 + lax.axis_index("c")
        base = wid * PER_W
        pltpu.sync_copy(idx_hbm.at[pl.ds(base, PER_W)], idx_v)

        def start_gather(c, p):
            pltpu.make_async_copy(
                flat_hbm.at[idx_v.at[pl.ds(c * CHUNK, CHUNK)]],
                bufs[p], gsems[p]).start()

        def wait_gather(p):
            pltpu.make_async_copy(
                flat_hbm.at[idx_v.at[pl.ds(0, CHUNK)]],
                bufs[p], gsems[p]).wait()

        def start_wb(c, p):
            pltpu.make_async_copy(
                bufs[p], out_hbm.at[pl.ds(base + c * CHUNK, CHUNK)],
                osems[p]).start()

        def wait_wb(p):
            pltpu.make_async_copy(
                bufs[p], out_hbm.at[pl.ds(base, CHUNK)], osems[p]).wait()

        for p in range(NBUF):
            start_gather(p, p)

        @pl.loop(0, NCHUNK, step=NBUF)
        def _(c):
            for p in range(NBUF):
                ch = c + p
                wait_gather(p)
                start_wb(ch, p)
                nxt = ch + NBUF

                @pl.when(nxt < NCHUNK)
                def _():
                    wait_wb(p)
                    start_gather(nxt, p)

        for p in range(NBUF):
            wait_wb(p)

    return gk(flat, idx)


def kernel(querys, prompts_key, prompts):
    pk_pad = jnp.pad(prompts_key, ((0, POOL_PAD - POOL), (0, 0)))
    sim_topk, idx8, loss = _topk_call(querys, pk_pad)
    flat = prompts.reshape(POOL * PROMPT_LEN, HIDDEN)
    gathered = _sc_gather(flat, idx8.reshape(NIDX))
    selected = gathered.reshape(BATCH, TOPK, PROMPT_LEN, HIDDEN)
    return selected, sim_topk, loss.reshape(())
